# Initial kernel scaffold; baseline (speedup 1.0000x reference)
import jax
import jax.numpy as jnp
from jax.experimental import pallas as pl


def kernel(node_features, node_attrs, edge_embedding, edge_attrs, edge_index, W1, Wr1, br1, Wr2, W2, Wu, W3, Wsc):
    raise NotImplementedError("write your pallas kernel here")


# speedup vs baseline: 2.5189x; 2.5189x over previous
import functools

import jax
import jax.numpy as jnp
from jax import lax
from jax.experimental import pallas as pl
from jax.experimental.pallas import tpu as pltpu
from jax.experimental.pallas import tpu_sc as plsc

_AVG_NEIGH = 16.0


def _mm_body(a_ref, b_ref, o_ref):
    o_ref[...] = jnp.dot(a_ref[...], b_ref[...],
                         preferred_element_type=jnp.float32)


def _linear(a, b, block_rows):
    n, k = a.shape
    m = b.shape[1]
    return pl.pallas_call(
        _mm_body,
        grid=(n // block_rows,),
        in_specs=[pl.BlockSpec((block_rows, k), lambda i: (i, 0)),
                  pl.BlockSpec((k, m), lambda i: (0, 0))],
        out_specs=pl.BlockSpec((block_rows, m), lambda i: (i, 0)),
        out_shape=jax.ShapeDtypeStruct((n, m), jnp.float32),
    )(a, b)


def _edge_weight_body(ee_ref, ea_ref, wr1_ref, br1_ref, wr2v_ref, o_ref):
    dea = wr2v_ref.shape[0]
    z = jnp.dot(ee_ref[...], wr1_ref[...],
                preferred_element_type=jnp.float32) + br1_ref[...]
    h = z * (1.0 / (1.0 + jnp.exp(-z)))
    acc = jnp.dot(h, wr2v_ref[0],
                  preferred_element_type=jnp.float32) * ea_ref[:, 0:1]
    for v in range(1, dea):
        acc += jnp.dot(h, wr2v_ref[v],
                       preferred_element_type=jnp.float32) * ea_ref[:, v:v + 1]
    o_ref[...] = acc


def _edge_weights(ee, ea, wr1, br1, wr2v, block_rows):
    e_p, de = ee.shape
    dea = ea.shape[1]
    h = wr1.shape[1]
    d = wr2v.shape[2]
    return pl.pallas_call(
        _edge_weight_body,
        grid=(e_p // block_rows,),
        in_specs=[pl.BlockSpec((block_rows, de), lambda i: (i, 0)),
                  pl.BlockSpec((block_rows, dea), lambda i: (i, 0)),
                  pl.BlockSpec((de, h), lambda i: (0, 0)),
                  pl.BlockSpec((1, h), lambda i: (0, 0)),
                  pl.BlockSpec((dea, h, d), lambda i: (0, 0, 0))],
        out_specs=pl.BlockSpec((block_rows, d), lambda i: (i, 0)),
        out_shape=jax.ShapeDtypeStruct((e_p, d), jnp.float32),
    )(ee, ea, wr1, br1, wr2v)


def _make_sc_scatter(n_pad, d, chunks_per_tile, ch):
    info = plsc.get_sparse_core_info()
    nc, ns = info.num_cores, info.num_subcores
    rows_per_tile = n_pad // ns
    lanes = d // 16
    mesh = plsc.VectorSubcoreMesh(core_axis_name="c", subcore_axis_name="s")

    zfull, zrem = divmod(rows_per_tile, ch)

    @functools.partial(
        pl.kernel, mesh=mesh,
        out_type=jax.ShapeDtypeStruct((nc, n_pad, d), jnp.float32),
        scratch_types=[
            pltpu.VMEM((ch,), jnp.int32),
            pltpu.VMEM((ch,), jnp.int32),
            pltpu.VMEM((ch, d), jnp.float32),
            pltpu.VMEM((ch, d), jnp.float32),
            pltpu.VMEM_SHARED((n_pad, d), jnp.float32),
            pltpu.SemaphoreType.DMA,
        ],
    )
    def sck(x_hbm, w_hbm, src_hbm, dst_hbm, out_hbm,
            src_v, dst_v, rows_v, wbuf_v, msg_sh, sem):
        c = lax.axis_index("c")
        s = lax.axis_index("s")
        wid = s * nc + c

        zvec = jnp.zeros((16,), jnp.float32)

        def zrow(i, carry):
            for j in range(lanes):
                rows_v[i, pl.ds(j * 16, 16)] = zvec
            return carry

        lax.fori_loop(0, ch, zrow, 0)
        for z in range(zfull):
            pltpu.sync_copy(rows_v,
                            msg_sh.at[pl.ds(s * rows_per_tile + z * ch, ch)])
        if zrem:
            pltpu.sync_copy(
                rows_v.at[pl.ds(0, zrem)],
                msg_sh.at[pl.ds(s * rows_per_tile + zfull * ch, zrem)])
        plsc.subcore_barrier()

        def chunk(g, carry):
            base = wid * (chunks_per_tile * ch) + g * ch
            pltpu.sync_copy(src_hbm.at[pl.ds(base, ch)], src_v)
            pltpu.sync_copy(dst_hbm.at[pl.ds(base, ch)], dst_v)
            gat = pltpu.async_copy(x_hbm.at[src_v], rows_v, sem)
            pltpu.sync_copy(w_hbm.at[pl.ds(base, ch)], wbuf_v)
            gat.wait()

            def mrow(i, cc):
                for j in range(lanes):
                    sl = pl.ds(j * 16, 16)
                    rows_v[i, sl] = rows_v[i, sl] * wbuf_v[i, sl]
                return cc

            lax.fori_loop(0, ch, mrow, 0)
            pltpu.sync_copy(rows_v, msg_sh.at[dst_v], add=True)
            return carry

        lax.fori_loop(0, chunks_per_tile, chunk, 0)
        plsc.subcore_barrier()
        pltpu.sync_copy(msg_sh.at[pl.ds(s * rows_per_tile, rows_per_tile)],
                        out_hbm.at[c, pl.ds(s * rows_per_tile, rows_per_tile)])

    return sck


def _post_body(p_ref, na_ref, nf_ref, w2_ref, wut_ref, w3_ref, wsct_ref,
               o_ref):
    da = wsct_ref.shape[0]
    msg = (p_ref[0] + p_ref[1]) * (1.0 / (_AVG_NEIGH ** 0.5))
    t = jnp.dot(msg, w2_ref[...], preferred_element_type=jnp.float32)
    upd = t * jnp.dot(na_ref[...], wut_ref[...],
                      preferred_element_type=jnp.float32)
    out = jnp.dot(upd, w3_ref[...], preferred_element_type=jnp.float32)
    sc = jnp.dot(nf_ref[...], wsct_ref[0],
                 preferred_element_type=jnp.float32) * na_ref[:, 0:1]
    for v in range(1, da):
        sc += jnp.dot(nf_ref[...], wsct_ref[v],
                      preferred_element_type=jnp.float32) * na_ref[:, v:v + 1]
    o_ref[...] = out + sc


def _post(partials, na, nf, w2, wut, w3, wsct, block_rows):
    n, d = nf.shape
    da = na.shape[1]
    return pl.pallas_call(
        _post_body,
        grid=(n // block_rows,),
        in_specs=[pl.BlockSpec((2, block_rows, d), lambda i: (0, i, 0)),
                  pl.BlockSpec((block_rows, da), lambda i: (i, 0)),
                  pl.BlockSpec((block_rows, d), lambda i: (i, 0)),
                  pl.BlockSpec((d, d), lambda i: (0, 0)),
                  pl.BlockSpec((da, d), lambda i: (0, 0)),
                  pl.BlockSpec((d, d), lambda i: (0, 0)),
                  pl.BlockSpec((da, d, d), lambda i: (0, 0, 0))],
        out_specs=pl.BlockSpec((block_rows, d), lambda i: (i, 0)),
        out_shape=jax.ShapeDtypeStruct((n, d), jnp.float32),
    )(partials, na, nf, w2, wut, w3, wsct)


def kernel(node_features, node_attrs, edge_embedding, edge_attrs, edge_index,
           W1, Wr1, br1, Wr2, W2, Wu, W3, Wsc):
    n, d = node_features.shape
    da = node_attrs.shape[1]
    e, de = edge_embedding.shape
    dea = edge_attrs.shape[1]
    h = Wr1.shape[1]

    ch = 128
    n_tiles = 32
    per_tile = ch * n_tiles
    e_p = ((e + per_tile - 1) // per_tile) * per_tile
    chunks_per_tile = e_p // per_tile

    pad = e_p - e
    ee = jnp.pad(edge_embedding, ((0, pad), (0, 0)))
    ea = jnp.pad(edge_attrs, ((0, pad), (0, 0)))
    src = jnp.pad(edge_index[0], (0, pad))
    dst = jnp.pad(edge_index[1], (0, pad))

    wr2v = jnp.transpose(Wr2.reshape(h, d, dea), (2, 0, 1))
    wsct = jnp.transpose(Wsc, (2, 1, 0))
    wut = Wu.T

    n_pad = ((n + 127) // 128) * 128

    x = _linear(node_features, W1, block_rows=2000)
    weighted = _edge_weights(ee, ea, Wr1, br1[None, :], wr2v, block_rows=4096)
    partials = _make_sc_scatter(n_pad, d, chunks_per_tile, ch)(x, weighted,
                                                               src, dst)
    return _post(partials[:, :n], node_attrs, node_features, W2, wut, W3,
                 wsct, block_rows=2000)


# speedup vs baseline: 2.8294x; 1.1233x over previous
import functools

import jax
import jax.numpy as jnp
from jax import lax
from jax.experimental import pallas as pl
from jax.experimental.pallas import tpu as pltpu
from jax.experimental.pallas import tpu_sc as plsc

_AVG_NEIGH = 16.0


def _mm_body(a_ref, b_ref, o_ref):
    o_ref[...] = jnp.dot(a_ref[...], b_ref[...],
                         preferred_element_type=jnp.float32)


def _linear(a, b, block_rows):
    n, k = a.shape
    m = b.shape[1]
    return pl.pallas_call(
        _mm_body,
        grid=(n // block_rows,),
        in_specs=[pl.BlockSpec((block_rows, k), lambda i: (i, 0)),
                  pl.BlockSpec((k, m), lambda i: (0, 0))],
        out_specs=pl.BlockSpec((block_rows, m), lambda i: (i, 0)),
        out_shape=jax.ShapeDtypeStruct((n, m), jnp.float32),
    )(a, b)


def _edge_weight_body(ee_ref, ea_ref, wr1_ref, br1_ref, wr2v_ref, o_ref):
    dea = wr2v_ref.shape[0]
    z = jnp.dot(ee_ref[...], wr1_ref[...],
                preferred_element_type=jnp.float32) + br1_ref[...]
    h = z * (1.0 / (1.0 + jnp.exp(-z)))
    acc = jnp.dot(h, wr2v_ref[0],
                  preferred_element_type=jnp.float32) * ea_ref[:, 0:1]
    for v in range(1, dea):
        acc += jnp.dot(h, wr2v_ref[v],
                       preferred_element_type=jnp.float32) * ea_ref[:, v:v + 1]
    o_ref[...] = acc


def _edge_weights(ee, ea, wr1, br1, wr2v, block_rows):
    e_p, de = ee.shape
    dea = ea.shape[1]
    h = wr1.shape[1]
    d = wr2v.shape[2]
    return pl.pallas_call(
        _edge_weight_body,
        grid=(e_p // block_rows,),
        in_specs=[pl.BlockSpec((block_rows, de), lambda i: (i, 0)),
                  pl.BlockSpec((block_rows, dea), lambda i: (i, 0)),
                  pl.BlockSpec((de, h), lambda i: (0, 0)),
                  pl.BlockSpec((1, h), lambda i: (0, 0)),
                  pl.BlockSpec((dea, h, d), lambda i: (0, 0, 0))],
        out_specs=pl.BlockSpec((block_rows, d), lambda i: (i, 0)),
        out_shape=jax.ShapeDtypeStruct((e_p, d), jnp.float32),
    )(ee, ea, wr1, br1, wr2v)


def _make_sc_scatter(n_pad, d, chunks_per_tile, ch):
    info = plsc.get_sparse_core_info()
    nc, ns = info.num_cores, info.num_subcores
    rows_per_tile = n_pad // ns
    lanes = d // 16
    nch = chunks_per_tile
    mesh = plsc.VectorSubcoreMesh(core_axis_name="c", subcore_axis_name="s")

    zfull, zrem = divmod(rows_per_tile, ch)

    @functools.partial(
        pl.kernel, mesh=mesh,
        out_type=jax.ShapeDtypeStruct((nc, n_pad, d), jnp.float32),
        scratch_types=[
            pltpu.VMEM((nch, ch), jnp.int32),
            pltpu.VMEM((nch, ch), jnp.int32),
            pltpu.VMEM((2, ch, d), jnp.float32),
            pltpu.VMEM((ch, d), jnp.float32),
            pltpu.VMEM_SHARED((n_pad, d), jnp.float32),
            pltpu.SemaphoreType.DMA((2,)),
            pltpu.SemaphoreType.DMA,
            pltpu.SemaphoreType.DMA((2,)),
        ],
    )
    def sck(x_hbm, w_hbm, src_hbm, dst_hbm, out_hbm,
            src_v, dst_v, rows_v, wbuf_v, msg_sh, gsem, wsem, ssem):
        c = lax.axis_index("c")
        s = lax.axis_index("s")
        wid = s * nc + c

        pltpu.sync_copy(src_hbm.at[pl.ds(wid * nch, nch)], src_v)
        pltpu.sync_copy(dst_hbm.at[pl.ds(wid * nch, nch)], dst_v)

        zvec = jnp.zeros((16,), jnp.float32)

        def zrow(i, carry):
            for j in range(lanes):
                rows_v[0, i, pl.ds(j * 16, 16)] = zvec
            return carry

        lax.fori_loop(0, ch, zrow, 0)
        for z in range(zfull):
            pltpu.sync_copy(rows_v.at[0],
                            msg_sh.at[pl.ds(s * rows_per_tile + z * ch, ch)])
        if zrem:
            pltpu.sync_copy(
                rows_v.at[0, pl.ds(0, zrem)],
                msg_sh.at[pl.ds(s * rows_per_tile + zfull * ch, zrem)])
        plsc.subcore_barrier()

        def issue_gather(g, b):
            pltpu.async_copy(x_hbm.at[src_v.at[g]], rows_v.at[b], gsem.at[b])

        def issue_wload(g):
            base = (wid * nch + g) * ch
            pltpu.async_copy(w_hbm.at[pl.ds(base, ch)], wbuf_v, wsem)

        issue_gather(0, 0)
        issue_wload(0)

        def outer(t, carry):
            for b in range(2):
                g = 2 * t + b
                nb = 1 - b

                @pl.when(g >= 1)
                def _wait_prev_scatter():
                    pltpu.make_async_copy(
                        rows_v.at[nb], msg_sh.at[dst_v.at[0]],
                        ssem.at[nb]).wait()

                @pl.when(g + 1 < nch)
                def _prefetch():
                    issue_gather(g + 1, nb)

                pltpu.make_async_copy(x_hbm.at[src_v.at[0]], rows_v.at[b],
                                      gsem.at[b]).wait()
                pltpu.make_async_copy(w_hbm.at[pl.ds(0, ch)], wbuf_v,
                                      wsem).wait()

                @plsc.parallel_loop(0, ch, unroll=4)
                def _mul(i):
                    for j in range(lanes):
                        sl = pl.ds(j * 16, 16)
                        rows_v[b, i, sl] = rows_v[b, i, sl] * wbuf_v[i, sl]

                pltpu.async_copy(rows_v.at[b], msg_sh.at[dst_v.at[g]],
                                 ssem.at[b], add=True)

                @pl.when(g + 1 < nch)
                def _next_wload():
                    issue_wload(g + 1)
            return carry

        lax.fori_loop(0, nch // 2, outer, 0)
        pltpu.make_async_copy(rows_v.at[1], msg_sh.at[dst_v.at[0]],
                              ssem.at[1]).wait()
        plsc.subcore_barrier()
        pltpu.sync_copy(msg_sh.at[pl.ds(s * rows_per_tile, rows_per_tile)],
                        out_hbm.at[c, pl.ds(s * rows_per_tile, rows_per_tile)])

    return sck


def _post_body(p_ref, na_ref, nf_ref, w2_ref, wut_ref, w3_ref, wsct_ref,
               o_ref):
    da = wsct_ref.shape[0]
    msg = (p_ref[0] + p_ref[1]) * (1.0 / (_AVG_NEIGH ** 0.5))
    t = jnp.dot(msg, w2_ref[...], preferred_element_type=jnp.float32)
    upd = t * jnp.dot(na_ref[...], wut_ref[...],
                      preferred_element_type=jnp.float32)
    out = jnp.dot(upd, w3_ref[...], preferred_element_type=jnp.float32)
    sc = jnp.dot(nf_ref[...], wsct_ref[0],
                 preferred_element_type=jnp.float32) * na_ref[:, 0:1]
    for v in range(1, da):
        sc += jnp.dot(nf_ref[...], wsct_ref[v],
                      preferred_element_type=jnp.float32) * na_ref[:, v:v + 1]
    o_ref[...] = out + sc


def _post(partials, na, nf, w2, wut, w3, wsct, block_rows):
    n, d = nf.shape
    da = na.shape[1]
    return pl.pallas_call(
        _post_body,
        grid=(n // block_rows,),
        in_specs=[pl.BlockSpec((2, block_rows, d), lambda i: (0, i, 0)),
                  pl.BlockSpec((block_rows, da), lambda i: (i, 0)),
                  pl.BlockSpec((block_rows, d), lambda i: (i, 0)),
                  pl.BlockSpec((d, d), lambda i: (0, 0)),
                  pl.BlockSpec((da, d), lambda i: (0, 0)),
                  pl.BlockSpec((d, d), lambda i: (0, 0)),
                  pl.BlockSpec((da, d, d), lambda i: (0, 0, 0))],
        out_specs=pl.BlockSpec((block_rows, d), lambda i: (i, 0)),
        out_shape=jax.ShapeDtypeStruct((n, d), jnp.float32),
    )(partials, na, nf, w2, wut, w3, wsct)


def kernel(node_features, node_attrs, edge_embedding, edge_attrs, edge_index,
           W1, Wr1, br1, Wr2, W2, Wu, W3, Wsc):
    n, d = node_features.shape
    da = node_attrs.shape[1]
    e, de = edge_embedding.shape
    dea = edge_attrs.shape[1]
    h = Wr1.shape[1]

    ch = 64
    n_tiles = 32
    per_tile = 2 * ch * n_tiles
    e_p = ((e + per_tile - 1) // per_tile) * per_tile
    chunks_per_tile = e_p // (ch * n_tiles)

    pad = e_p - e
    ee = jnp.pad(edge_embedding, ((0, pad), (0, 0)))
    ea = jnp.pad(edge_attrs, ((0, pad), (0, 0)))
    src = jnp.pad(edge_index[0], (0, pad)).reshape(-1, ch)
    dst = jnp.pad(edge_index[1], (0, pad)).reshape(-1, ch)

    wr2v = jnp.transpose(Wr2.reshape(h, d, dea), (2, 0, 1))
    wsct = jnp.transpose(Wsc, (2, 1, 0))
    wut = Wu.T

    n_pad = ((n + 127) // 128) * 128

    x = _linear(node_features, W1, block_rows=2000)
    weighted = _edge_weights(ee, ea, Wr1, br1[None, :], wr2v, block_rows=4096)
    partials = _make_sc_scatter(n_pad, d, chunks_per_tile, ch)(x, weighted,
                                                               src, dst)
    return _post(partials[:, :n], node_attrs, node_features, W2, wut, W3,
                 wsct, block_rows=2000)


# speedup vs baseline: 3.3118x; 1.1705x over previous
import functools

import jax
import jax.numpy as jnp
from jax import lax
from jax.experimental import pallas as pl
from jax.experimental.pallas import tpu as pltpu
from jax.experimental.pallas import tpu_sc as plsc

_AVG_NEIGH = 16.0


def _mm_body(a_ref, b_ref, o_ref):
    o_ref[...] = jnp.dot(a_ref[...], b_ref[...],
                         preferred_element_type=jnp.float32)


def _linear(a, b, block_rows):
    n, k = a.shape
    m = b.shape[1]
    return pl.pallas_call(
        _mm_body,
        grid=(n // block_rows,),
        in_specs=[pl.BlockSpec((block_rows, k), lambda i: (i, 0)),
                  pl.BlockSpec((k, m), lambda i: (0, 0))],
        out_specs=pl.BlockSpec((block_rows, m), lambda i: (i, 0)),
        out_shape=jax.ShapeDtypeStruct((n, m), jnp.float32),
    )(a, b)


def _make_edge_weight_body(e, block_rows):
    def body(ee_ref, ea_ref, wr1_ref, br1_ref, wr2v_ref, o_ref):
        dea = wr2v_ref.shape[0]
        z = jnp.dot(ee_ref[...], wr1_ref[...],
                    preferred_element_type=jnp.float32) + br1_ref[...]
        h = z * (1.0 / (1.0 + jnp.exp(-z)))
        acc = jnp.dot(h, wr2v_ref[0],
                      preferred_element_type=jnp.float32) * ea_ref[:, 0:1]
        for v in range(1, dea):
            acc += jnp.dot(h, wr2v_ref[v],
                           preferred_element_type=jnp.float32) * ea_ref[:,
                                                                        v:v + 1]
        row = (pl.program_id(0) * block_rows
               + jax.lax.broadcasted_iota(jnp.int32, acc.shape, 0))
        o_ref[...] = jnp.where(row < e, acc, 0.0)
    return body


def _edge_weights(ee, ea, wr1, br1, wr2v, e_p, block_rows):
    e, de = ee.shape
    dea = ea.shape[1]
    h = wr1.shape[1]
    d = wr2v.shape[2]
    return pl.pallas_call(
        _make_edge_weight_body(e, block_rows),
        grid=(e_p // block_rows,),
        in_specs=[pl.BlockSpec((block_rows, de), lambda i: (i, 0)),
                  pl.BlockSpec((block_rows, dea), lambda i: (i, 0)),
                  pl.BlockSpec((de, h), lambda i: (0, 0)),
                  pl.BlockSpec((1, h), lambda i: (0, 0)),
                  pl.BlockSpec((dea, h, d), lambda i: (0, 0, 0))],
        out_specs=pl.BlockSpec((block_rows, d), lambda i: (i, 0)),
        out_shape=jax.ShapeDtypeStruct((e_p, d), jnp.float32),
    )(ee, ea, wr1, br1, wr2v)


def _make_sc_scatter(n_pad, d, chunks_per_tile, ch):
    info = plsc.get_sparse_core_info()
    nc, ns = info.num_cores, info.num_subcores
    rows_per_tile = n_pad // ns
    lanes = d // 16
    nch = chunks_per_tile
    mesh = plsc.VectorSubcoreMesh(core_axis_name="c", subcore_axis_name="s")

    zfull, zrem = divmod(rows_per_tile, ch)

    @functools.partial(
        pl.kernel, mesh=mesh,
        out_type=jax.ShapeDtypeStruct((nc, n_pad, d), jnp.float32),
        scratch_types=[
            pltpu.VMEM((nch, ch), jnp.int32),
            pltpu.VMEM((nch, ch), jnp.int32),
            pltpu.VMEM((2, ch, d), jnp.float32),
            pltpu.VMEM((ch, d), jnp.float32),
            pltpu.VMEM_SHARED((n_pad, d), jnp.float32),
            pltpu.SemaphoreType.DMA((2,)),
            pltpu.SemaphoreType.DMA,
            pltpu.SemaphoreType.DMA((2,)),
        ],
    )
    def sck(x_hbm, w_hbm, src_hbm, dst_hbm, out_hbm,
            src_v, dst_v, rows_v, wbuf_v, msg_sh, gsem, wsem, ssem):
        c = lax.axis_index("c")
        s = lax.axis_index("s")
        wid = s * nc + c

        pltpu.sync_copy(src_hbm.at[pl.ds(wid * nch, nch)], src_v)
        pltpu.sync_copy(dst_hbm.at[pl.ds(wid * nch, nch)], dst_v)

        zvec = jnp.zeros((16,), jnp.float32)

        def zrow(i, carry):
            for j in range(lanes):
                rows_v[0, i, pl.ds(j * 16, 16)] = zvec
            return carry

        lax.fori_loop(0, ch, zrow, 0)
        for z in range(zfull):
            pltpu.sync_copy(rows_v.at[0],
                            msg_sh.at[pl.ds(s * rows_per_tile + z * ch, ch)])
        if zrem:
            pltpu.sync_copy(
                rows_v.at[0, pl.ds(0, zrem)],
                msg_sh.at[pl.ds(s * rows_per_tile + zfull * ch, zrem)])
        plsc.subcore_barrier()

        def issue_gather(g, b):
            pltpu.async_copy(x_hbm.at[src_v.at[g]], rows_v.at[b], gsem.at[b])

        def issue_wload(g):
            base = (wid * nch + g) * ch
            pltpu.async_copy(w_hbm.at[pl.ds(base, ch)], wbuf_v, wsem)

        issue_gather(0, 0)
        issue_wload(0)

        def outer(t, carry):
            for b in range(2):
                g = 2 * t + b
                nb = 1 - b

                @pl.when(g >= 1)
                def _wait_prev_scatter():
                    pltpu.make_async_copy(
                        rows_v.at[nb], msg_sh.at[dst_v.at[0]],
                        ssem.at[nb]).wait()

                @pl.when(g + 1 < nch)
                def _prefetch():
                    issue_gather(g + 1, nb)

                pltpu.make_async_copy(x_hbm.at[src_v.at[0]], rows_v.at[b],
                                      gsem.at[b]).wait()
                pltpu.make_async_copy(w_hbm.at[pl.ds(0, ch)], wbuf_v,
                                      wsem).wait()

                @plsc.parallel_loop(0, ch, unroll=4)
                def _mul(i):
                    for j in range(lanes):
                        sl = pl.ds(j * 16, 16)
                        rows_v[b, i, sl] = rows_v[b, i, sl] * wbuf_v[i, sl]

                pltpu.async_copy(rows_v.at[b], msg_sh.at[dst_v.at[g]],
                                 ssem.at[b], add=True)

                @pl.when(g + 1 < nch)
                def _next_wload():
                    issue_wload(g + 1)
            return carry

        lax.fori_loop(0, nch // 2, outer, 0)
        pltpu.make_async_copy(rows_v.at[1], msg_sh.at[dst_v.at[0]],
                              ssem.at[1]).wait()
        plsc.subcore_barrier()
        pltpu.sync_copy(msg_sh.at[pl.ds(s * rows_per_tile, rows_per_tile)],
                        out_hbm.at[c, pl.ds(s * rows_per_tile, rows_per_tile)])

    return sck


def _post_body(p_ref, na_ref, nf_ref, w2_ref, wut_ref, w3_ref, wsct_ref,
               o_ref):
    da = wsct_ref.shape[0]
    msg = (p_ref[0] + p_ref[1]) * (1.0 / (_AVG_NEIGH ** 0.5))
    t = jnp.dot(msg, w2_ref[...], preferred_element_type=jnp.float32)
    upd = t * jnp.dot(na_ref[...], wut_ref[...],
                      preferred_element_type=jnp.float32)
    out = jnp.dot(upd, w3_ref[...], preferred_element_type=jnp.float32)
    sc = jnp.dot(nf_ref[...], wsct_ref[0],
                 preferred_element_type=jnp.float32) * na_ref[:, 0:1]
    for v in range(1, da):
        sc += jnp.dot(nf_ref[...], wsct_ref[v],
                      preferred_element_type=jnp.float32) * na_ref[:, v:v + 1]
    o_ref[...] = out + sc


def _post(partials, na, nf, w2, wut, w3, wsct, block_rows):
    n, d = nf.shape
    da = na.shape[1]
    return pl.pallas_call(
        _post_body,
        grid=(n // block_rows,),
        in_specs=[pl.BlockSpec((2, block_rows, d), lambda i: (0, i, 0)),
                  pl.BlockSpec((block_rows, da), lambda i: (i, 0)),
                  pl.BlockSpec((block_rows, d), lambda i: (i, 0)),
                  pl.BlockSpec((d, d), lambda i: (0, 0)),
                  pl.BlockSpec((da, d), lambda i: (0, 0)),
                  pl.BlockSpec((d, d), lambda i: (0, 0)),
                  pl.BlockSpec((da, d, d), lambda i: (0, 0, 0))],
        out_specs=pl.BlockSpec((block_rows, d), lambda i: (i, 0)),
        out_shape=jax.ShapeDtypeStruct((n, d), jnp.float32),
    )(partials, na, nf, w2, wut, w3, wsct)


def kernel(node_features, node_attrs, edge_embedding, edge_attrs, edge_index,
           W1, Wr1, br1, Wr2, W2, Wu, W3, Wsc):
    n, d = node_features.shape
    da = node_attrs.shape[1]
    e, de = edge_embedding.shape
    dea = edge_attrs.shape[1]
    h = Wr1.shape[1]

    ch = 64
    n_tiles = 32
    per_tile = 2 * ch * n_tiles
    e_p = ((e + per_tile - 1) // per_tile) * per_tile
    chunks_per_tile = e_p // (ch * n_tiles)

    pad = e_p - e
    src = jnp.pad(edge_index[0], (0, pad)).reshape(-1, ch)
    dst = jnp.pad(edge_index[1], (0, pad)).reshape(-1, ch)

    wr2v = jnp.transpose(Wr2.reshape(h, d, dea), (2, 0, 1))
    wsct = jnp.transpose(Wsc, (2, 1, 0))
    wut = Wu.T

    n_pad = ((n + 127) // 128) * 128

    x = _linear(node_features, W1, block_rows=2000)
    weighted = _edge_weights(edge_embedding, edge_attrs, Wr1, br1[None, :],
                             wr2v, e_p, block_rows=4096)
    partials = _make_sc_scatter(n_pad, d, chunks_per_tile, ch)(x, weighted,
                                                               src, dst)
    return _post(partials, node_attrs, node_features, W2, wut, W3,
                 wsct, block_rows=2000)


# speedup vs baseline: 4.6283x; 1.3975x over previous
import functools

import jax
import jax.numpy as jnp
from jax import lax
from jax.experimental import pallas as pl
from jax.experimental.pallas import tpu as pltpu
from jax.experimental.pallas import tpu_sc as plsc

_AVG_NEIGH = 16.0


def _mm_body(a_ref, b_ref, o_ref):
    o_ref[...] = jnp.dot(a_ref[...], b_ref[...],
                         preferred_element_type=jnp.float32)


def _linear(a, b, block_rows):
    n, k = a.shape
    m = b.shape[1]
    return pl.pallas_call(
        _mm_body,
        grid=(n // block_rows,),
        in_specs=[pl.BlockSpec((block_rows, k), lambda i: (i, 0)),
                  pl.BlockSpec((k, m), lambda i: (0, 0))],
        out_specs=pl.BlockSpec((block_rows, m), lambda i: (i, 0)),
        out_shape=jax.ShapeDtypeStruct((n, m), jnp.float32),
    )(a, b)


def _make_edge_weight_body(e, block_rows):
    def body(eet_ref, eat_ref, wr1_ref, br1_ref, wr2p_ref, o_ref):
        z = jax.lax.dot_general(wr1_ref[...], eet_ref[...],
                                (((0,), (0,)), ((), ())),
                                preferred_element_type=jnp.float32)
        z = z + br1_ref[...]
        h = z * (1.0 / (1.0 + jnp.exp(-z)))
        hh, dea = h.shape[0], eat_ref.shape[0]
        tmp = (h[:, None, :] * eat_ref[...][None, :, :]).reshape(
            hh * dea, h.shape[1])
        acc = jax.lax.dot_general(tmp, wr2p_ref[...],
                                  (((0,), (0,)), ((), ())),
                                  preferred_element_type=jnp.float32)
        row = (pl.program_id(0) * block_rows
               + jax.lax.broadcasted_iota(jnp.int32, acc.shape, 0))
        o_ref[...] = jnp.where(row < e, acc, 0.0)
    return body


def _edge_weights(eet, eat, wr1, br1, wr2p, e_p, block_rows):
    de, e = eet.shape
    dea = eat.shape[0]
    h = wr1.shape[1]
    d = wr2p.shape[1]
    return pl.pallas_call(
        _make_edge_weight_body(e, block_rows),
        grid=(e_p // block_rows,),
        in_specs=[pl.BlockSpec((de, block_rows), lambda i: (0, i)),
                  pl.BlockSpec((dea, block_rows), lambda i: (0, i)),
                  pl.BlockSpec((de, h), lambda i: (0, 0)),
                  pl.BlockSpec((h, 1), lambda i: (0, 0)),
                  pl.BlockSpec((h * dea, d), lambda i: (0, 0))],
        out_specs=pl.BlockSpec((block_rows, d), lambda i: (i, 0)),
        out_shape=jax.ShapeDtypeStruct((e_p, d), jnp.float32),
    )(eet, eat, wr1, br1, wr2p)


def _make_sc_scatter(n_pad, d, chunks_per_tile, ch):
    info = plsc.get_sparse_core_info()
    nc, ns = info.num_cores, info.num_subcores
    rows_per_tile = n_pad // ns
    lanes = d // 16
    nch = chunks_per_tile
    mesh = plsc.VectorSubcoreMesh(core_axis_name="c", subcore_axis_name="s")

    zfull, zrem = divmod(rows_per_tile, ch)

    @functools.partial(
        pl.kernel, mesh=mesh,
        out_type=jax.ShapeDtypeStruct((nc, n_pad, d), jnp.float32),
        scratch_types=[
            pltpu.VMEM((nch, ch), jnp.int32),
            pltpu.VMEM((nch, ch), jnp.int32),
            pltpu.VMEM((2, ch, d), jnp.float32),
            pltpu.VMEM((ch, d), jnp.float32),
            pltpu.VMEM_SHARED((n_pad, d), jnp.float32),
            pltpu.SemaphoreType.DMA((2,)),
            pltpu.SemaphoreType.DMA,
            pltpu.SemaphoreType.DMA((2,)),
        ],
    )
    def sck(x_hbm, w_hbm, src_hbm, dst_hbm, out_hbm,
            src_v, dst_v, rows_v, wbuf_v, msg_sh, gsem, wsem, ssem):
        c = lax.axis_index("c")
        s = lax.axis_index("s")
        wid = s * nc + c

        pltpu.sync_copy(src_hbm.at[pl.ds(wid * nch, nch)], src_v)
        pltpu.sync_copy(dst_hbm.at[pl.ds(wid * nch, nch)], dst_v)

        zvec = jnp.zeros((16,), jnp.float32)

        def zrow(i, carry):
            for j in range(lanes):
                rows_v[0, i, pl.ds(j * 16, 16)] = zvec
            return carry

        lax.fori_loop(0, ch, zrow, 0)
        for z in range(zfull):
            pltpu.sync_copy(rows_v.at[0],
                            msg_sh.at[pl.ds(s * rows_per_tile + z * ch, ch)])
        if zrem:
            pltpu.sync_copy(
                rows_v.at[0, pl.ds(0, zrem)],
                msg_sh.at[pl.ds(s * rows_per_tile + zfull * ch, zrem)])
        plsc.subcore_barrier()

        def issue_gather(g, b):
            pltpu.async_copy(x_hbm.at[src_v.at[g]], rows_v.at[b], gsem.at[b])

        def issue_wload(g):
            base = (wid * nch + g) * ch
            pltpu.async_copy(w_hbm.at[pl.ds(base, ch)], wbuf_v, wsem)

        issue_gather(0, 0)
        issue_wload(0)

        def outer(t, carry):
            for b in range(2):
                g = 2 * t + b
                nb = 1 - b

                @pl.when(g >= 1)
                def _wait_prev_scatter():
                    pltpu.make_async_copy(
                        rows_v.at[nb], msg_sh.at[dst_v.at[0]],
                        ssem.at[nb]).wait()

                @pl.when(g + 1 < nch)
                def _prefetch():
                    issue_gather(g + 1, nb)

                pltpu.make_async_copy(x_hbm.at[src_v.at[0]], rows_v.at[b],
                                      gsem.at[b]).wait()
                pltpu.make_async_copy(w_hbm.at[pl.ds(0, ch)], wbuf_v,
                                      wsem).wait()

                @plsc.parallel_loop(0, ch, unroll=4)
                def _mul(i):
                    for j in range(lanes):
                        sl = pl.ds(j * 16, 16)
                        rows_v[b, i, sl] = rows_v[b, i, sl] * wbuf_v[i, sl]

                pltpu.async_copy(rows_v.at[b], msg_sh.at[dst_v.at[g]],
                                 ssem.at[b], add=True)

                @pl.when(g + 1 < nch)
                def _next_wload():
                    issue_wload(g + 1)
            return carry

        lax.fori_loop(0, nch // 2, outer, 0)
        pltpu.make_async_copy(rows_v.at[1], msg_sh.at[dst_v.at[0]],
                              ssem.at[1]).wait()
        plsc.subcore_barrier()
        pltpu.sync_copy(msg_sh.at[pl.ds(s * rows_per_tile, rows_per_tile)],
                        out_hbm.at[c, pl.ds(s * rows_per_tile, rows_per_tile)])

    return sck


def _selfconn_body(na_ref, nf_ref, wsct_ref, o_ref):
    da = wsct_ref.shape[0]
    sc = jnp.dot(nf_ref[...], wsct_ref[0],
                 preferred_element_type=jnp.float32) * na_ref[:, 0:1]
    for v in range(1, da):
        sc += jnp.dot(nf_ref[...], wsct_ref[v],
                      preferred_element_type=jnp.float32) * na_ref[:, v:v + 1]
    o_ref[...] = sc


def _selfconn(na, nf, wsct, block_rows):
    n, d = nf.shape
    da = na.shape[1]
    return pl.pallas_call(
        _selfconn_body,
        grid=(n // block_rows,),
        in_specs=[pl.BlockSpec((block_rows, da), lambda i: (i, 0)),
                  pl.BlockSpec((block_rows, d), lambda i: (i, 0)),
                  pl.BlockSpec((da, d, d), lambda i: (0, 0, 0))],
        out_specs=pl.BlockSpec((block_rows, d), lambda i: (i, 0)),
        out_shape=jax.ShapeDtypeStruct((n, d), jnp.float32),
    )(na, nf, wsct)


def _post_body(p_ref, na_ref, sc_ref, w2_ref, wut_ref, w3_ref, o_ref):
    msg = (p_ref[0] + p_ref[1]) * (1.0 / (_AVG_NEIGH ** 0.5))
    t = jnp.dot(msg, w2_ref[...], preferred_element_type=jnp.float32)
    upd = t * jnp.dot(na_ref[...], wut_ref[...],
                      preferred_element_type=jnp.float32)
    out = jnp.dot(upd, w3_ref[...], preferred_element_type=jnp.float32)
    o_ref[...] = out + sc_ref[...]


def _post(partials, na, sc, w2, wut, w3, block_rows):
    n, d = sc.shape
    da = na.shape[1]
    return pl.pallas_call(
        _post_body,
        grid=(n // block_rows,),
        in_specs=[pl.BlockSpec((2, block_rows, d), lambda i: (0, i, 0)),
                  pl.BlockSpec((block_rows, da), lambda i: (i, 0)),
                  pl.BlockSpec((block_rows, d), lambda i: (i, 0)),
                  pl.BlockSpec((d, d), lambda i: (0, 0)),
                  pl.BlockSpec((da, d), lambda i: (0, 0)),
                  pl.BlockSpec((d, d), lambda i: (0, 0))],
        out_specs=pl.BlockSpec((block_rows, d), lambda i: (i, 0)),
        out_shape=jax.ShapeDtypeStruct((n, d), jnp.float32),
    )(partials, na, sc, w2, wut, w3)


def kernel(node_features, node_attrs, edge_embedding, edge_attrs, edge_index,
           W1, Wr1, br1, Wr2, W2, Wu, W3, Wsc):
    n, d = node_features.shape
    da = node_attrs.shape[1]
    e, de = edge_embedding.shape
    dea = edge_attrs.shape[1]
    h = Wr1.shape[1]

    ch = 64
    n_tiles = 32
    per_tile = 2 * ch * n_tiles
    e_p = ((e + per_tile - 1) // per_tile) * per_tile
    chunks_per_tile = e_p // (ch * n_tiles)

    pad = e_p - e
    src = jnp.pad(edge_index[0], (0, pad)).reshape(-1, ch)
    dst = jnp.pad(edge_index[1], (0, pad)).reshape(-1, ch)

    wr2p = jnp.transpose(Wr2.reshape(h, d, dea), (0, 2, 1)).reshape(h * dea, d)
    wsct = jnp.transpose(Wsc, (2, 1, 0))
    wut = Wu.T

    n_pad = ((n + 127) // 128) * 128

    x = _linear(node_features, W1, block_rows=2000)
    weighted = _edge_weights(edge_embedding.T, edge_attrs.T, Wr1,
                             br1[:, None], wr2p, e_p, block_rows=4096)
    partials = _make_sc_scatter(n_pad, d, chunks_per_tile, ch)(x, weighted,
                                                               src, dst)
    sc = _selfconn(node_attrs, node_features, wsct, block_rows=2000)
    return _post(partials, node_attrs, sc, W2, wut, W3, block_rows=2000)


# speedup vs baseline: 4.7063x; 1.0169x over previous
import functools

import jax
import jax.numpy as jnp
from jax import lax
from jax.experimental import pallas as pl
from jax.experimental.pallas import tpu as pltpu
from jax.experimental.pallas import tpu_sc as plsc

_AVG_NEIGH = 16.0


def _mm_body(a_ref, b_ref, o_ref):
    o_ref[...] = jnp.dot(a_ref[...], b_ref[...],
                         preferred_element_type=jnp.float32)


def _linear(a, b, block_rows):
    n, k = a.shape
    m = b.shape[1]
    return pl.pallas_call(
        _mm_body,
        grid=(n // block_rows,),
        in_specs=[pl.BlockSpec((block_rows, k), lambda i: (i, 0)),
                  pl.BlockSpec((k, m), lambda i: (0, 0))],
        out_specs=pl.BlockSpec((block_rows, m), lambda i: (i, 0)),
        out_shape=jax.ShapeDtypeStruct((n, m), jnp.float32),
    )(a, b)


def _make_edge_weight_body(e, block_rows):
    def body(eet_ref, eat_ref, wr1_ref, br1_ref, wr2p_ref, o_ref):
        z = jax.lax.dot_general(wr1_ref[...], eet_ref[...],
                                (((0,), (0,)), ((), ())),
                                preferred_element_type=jnp.float32)
        z = z + br1_ref[...]
        h = z * (1.0 / (1.0 + jnp.exp(-z)))
        hh, dea = h.shape[0], eat_ref.shape[0]
        tmp = (h[:, None, :] * eat_ref[...][None, :, :]).reshape(
            hh * dea, h.shape[1])
        acc = jax.lax.dot_general(tmp, wr2p_ref[...],
                                  (((0,), (0,)), ((), ())),
                                  preferred_element_type=jnp.float32)
        row = (pl.program_id(0) * block_rows
               + jax.lax.broadcasted_iota(jnp.int32, acc.shape, 0))
        o_ref[...] = jnp.where(row < e, acc, 0.0)
    return body


def _edge_weights(eet, eat, wr1, br1, wr2p, e_p, block_rows):
    de, e = eet.shape
    dea = eat.shape[0]
    h = wr1.shape[1]
    d = wr2p.shape[1]
    return pl.pallas_call(
        _make_edge_weight_body(e, block_rows),
        grid=(e_p // block_rows,),
        in_specs=[pl.BlockSpec((de, block_rows), lambda i: (0, i)),
                  pl.BlockSpec((dea, block_rows), lambda i: (0, i)),
                  pl.BlockSpec((de, h), lambda i: (0, 0)),
                  pl.BlockSpec((h, 1), lambda i: (0, 0)),
                  pl.BlockSpec((h * dea, d), lambda i: (0, 0))],
        out_specs=pl.BlockSpec((block_rows, d), lambda i: (i, 0)),
        out_shape=jax.ShapeDtypeStruct((e_p, d), jnp.float32),
    )(eet, eat, wr1, br1, wr2p)


def _make_sc_scatter(n_pad, d, q0, q1, ch):
    info = plsc.get_sparse_core_info()
    nc, ns = info.num_cores, info.num_subcores
    rows_per_tile = n_pad // ns
    lanes = d // 16
    nch = max(q0, q1)
    mesh = plsc.VectorSubcoreMesh(core_axis_name="c", subcore_axis_name="s")

    zfull, zrem = divmod(rows_per_tile, ch)

    @functools.partial(
        pl.kernel, mesh=mesh,
        out_type=jax.ShapeDtypeStruct((nc, n_pad, d), jnp.float32),
        scratch_types=[
            pltpu.VMEM((nch, ch), jnp.int32),
            pltpu.VMEM((nch, ch), jnp.int32),
            pltpu.VMEM((2, ch, d), jnp.float32),
            pltpu.VMEM((ch, d), jnp.float32),
            pltpu.VMEM_SHARED((n_pad, d), jnp.float32),
            pltpu.SemaphoreType.DMA((2,)),
            pltpu.SemaphoreType.DMA,
            pltpu.SemaphoreType.DMA((2,)),
        ],
    )
    def sck(x_hbm, w_hbm, src_hbm, dst_hbm, out_hbm,
            src_v, dst_v, rows_v, wbuf_v, msg_sh, gsem, wsem, ssem):
        c = lax.axis_index("c")
        s = lax.axis_index("s")
        qc = jnp.where(c == 0, q0, q1)
        chunk0 = jnp.where(c == 0, s * q0, ns * q0 + s * q1)

        pltpu.sync_copy(src_hbm.at[pl.ds(chunk0, nch)], src_v)
        pltpu.sync_copy(dst_hbm.at[pl.ds(chunk0, nch)], dst_v)

        zvec = jnp.zeros((16,), jnp.float32)

        def zrow(i, carry):
            for j in range(lanes):
                rows_v[0, i, pl.ds(j * 16, 16)] = zvec
            return carry

        lax.fori_loop(0, ch, zrow, 0)
        for z in range(zfull):
            pltpu.sync_copy(rows_v.at[0],
                            msg_sh.at[pl.ds(s * rows_per_tile + z * ch, ch)])
        if zrem:
            pltpu.sync_copy(
                rows_v.at[0, pl.ds(0, zrem)],
                msg_sh.at[pl.ds(s * rows_per_tile + zfull * ch, zrem)])
        plsc.subcore_barrier()

        def issue_gather(g, b):
            pltpu.async_copy(x_hbm.at[src_v.at[g]], rows_v.at[b], gsem.at[b])

        def issue_wload(g):
            base = (chunk0 + g) * ch
            pltpu.async_copy(w_hbm.at[pl.ds(base, ch)], wbuf_v, wsem)

        issue_gather(0, 0)
        issue_wload(0)

        def outer(t, carry):
            for b in range(2):
                g = 2 * t + b
                nb = 1 - b

                @pl.when(g >= 1)
                def _wait_prev_scatter():
                    pltpu.make_async_copy(
                        rows_v.at[nb], msg_sh.at[dst_v.at[0]],
                        ssem.at[nb]).wait()

                @pl.when(g + 1 < qc)
                def _prefetch():
                    issue_gather(g + 1, nb)

                pltpu.make_async_copy(x_hbm.at[src_v.at[0]], rows_v.at[b],
                                      gsem.at[b]).wait()
                pltpu.make_async_copy(w_hbm.at[pl.ds(0, ch)], wbuf_v,
                                      wsem).wait()

                @plsc.parallel_loop(0, ch, unroll=4)
                def _mul(i):
                    for j in range(lanes):
                        sl = pl.ds(j * 16, 16)
                        rows_v[b, i, sl] = rows_v[b, i, sl] * wbuf_v[i, sl]

                pltpu.async_copy(rows_v.at[b], msg_sh.at[dst_v.at[g]],
                                 ssem.at[b], add=True)

                @pl.when(g + 1 < qc)
                def _next_wload():
                    issue_wload(g + 1)
            return carry

        lax.fori_loop(0, qc // 2, outer, 0)
        pltpu.make_async_copy(rows_v.at[1], msg_sh.at[dst_v.at[0]],
                              ssem.at[1]).wait()
        plsc.subcore_barrier()
        pltpu.sync_copy(msg_sh.at[pl.ds(s * rows_per_tile, rows_per_tile)],
                        out_hbm.at[c, pl.ds(s * rows_per_tile, rows_per_tile)])

    return sck


def _selfconn_body(na_ref, nf_ref, wsct_ref, o_ref):
    da = wsct_ref.shape[0]
    sc = jnp.dot(nf_ref[...], wsct_ref[0],
                 preferred_element_type=jnp.float32) * na_ref[:, 0:1]
    for v in range(1, da):
        sc += jnp.dot(nf_ref[...], wsct_ref[v],
                      preferred_element_type=jnp.float32) * na_ref[:, v:v + 1]
    o_ref[...] = sc


def _selfconn(na, nf, wsct, block_rows):
    n, d = nf.shape
    da = na.shape[1]
    return pl.pallas_call(
        _selfconn_body,
        grid=(n // block_rows,),
        in_specs=[pl.BlockSpec((block_rows, da), lambda i: (i, 0)),
                  pl.BlockSpec((block_rows, d), lambda i: (i, 0)),
                  pl.BlockSpec((da, d, d), lambda i: (0, 0, 0))],
        out_specs=pl.BlockSpec((block_rows, d), lambda i: (i, 0)),
        out_shape=jax.ShapeDtypeStruct((n, d), jnp.float32),
    )(na, nf, wsct)


def _post_body(p_ref, na_ref, sc_ref, w2_ref, wut_ref, w3_ref, o_ref):
    msg = (p_ref[0] + p_ref[1]) * (1.0 / (_AVG_NEIGH ** 0.5))
    t = jnp.dot(msg, w2_ref[...], preferred_element_type=jnp.float32)
    upd = t * jnp.dot(na_ref[...], wut_ref[...],
                      preferred_element_type=jnp.float32)
    out = jnp.dot(upd, w3_ref[...], preferred_element_type=jnp.float32)
    o_ref[...] = out + sc_ref[...]


def _post(partials, na, sc, w2, wut, w3, block_rows):
    n, d = sc.shape
    da = na.shape[1]
    return pl.pallas_call(
        _post_body,
        grid=(n // block_rows,),
        in_specs=[pl.BlockSpec((2, block_rows, d), lambda i: (0, i, 0)),
                  pl.BlockSpec((block_rows, da), lambda i: (i, 0)),
                  pl.BlockSpec((block_rows, d), lambda i: (i, 0)),
                  pl.BlockSpec((d, d), lambda i: (0, 0)),
                  pl.BlockSpec((da, d), lambda i: (0, 0)),
                  pl.BlockSpec((d, d), lambda i: (0, 0))],
        out_specs=pl.BlockSpec((block_rows, d), lambda i: (i, 0)),
        out_shape=jax.ShapeDtypeStruct((n, d), jnp.float32),
    )(partials, na, sc, w2, wut, w3)


def kernel(node_features, node_attrs, edge_embedding, edge_attrs, edge_index,
           W1, Wr1, br1, Wr2, W2, Wu, W3, Wsc):
    n, d = node_features.shape
    da = node_attrs.shape[1]
    e, de = edge_embedding.shape
    dea = edge_attrs.shape[1]
    h = Wr1.shape[1]

    ch = 64
    ns = 16
    grain = 2 * 2 * ch * ns
    e_p = ((e + grain - 1) // grain) * grain
    qsum = e_p // (ch * ns)
    q0 = max(2, int(round(qsum * 0.60 / 2)) * 2)
    q1 = qsum - q0

    pad = e_p - e
    src = jnp.pad(edge_index[0], (0, pad)).reshape(-1, ch)
    dst = jnp.pad(edge_index[1], (0, pad)).reshape(-1, ch)
    extra = max(0, ns * q0 + (ns - 1) * q1 + max(q0, q1) - src.shape[0])
    if extra:
        src = jnp.pad(src, ((0, extra), (0, 0)))
        dst = jnp.pad(dst, ((0, extra), (0, 0)))

    wr2p = jnp.transpose(Wr2.reshape(h, d, dea), (0, 2, 1)).reshape(h * dea, d)
    wsct = jnp.transpose(Wsc, (2, 1, 0))
    wut = Wu.T

    n_pad = ((n + 127) // 128) * 128

    x = _linear(node_features, W1, block_rows=2000)
    weighted = _edge_weights(edge_embedding.T, edge_attrs.T, Wr1,
                             br1[:, None], wr2p, e_p, block_rows=4096)
    partials = _make_sc_scatter(n_pad, d, q0, q1, ch)(x, weighted, src, dst)
    sc = _selfconn(node_attrs, node_features, wsct, block_rows=2000)
    return _post(partials, node_attrs, sc, W2, wut, W3, block_rows=2000)
